# baseline (speedup 1.0000x reference)
import functools
import math

import jax
import jax.numpy as jnp
from jax.experimental import pallas as pl
from jax.experimental.pallas import tpu as pltpu

_NUM_HEADS = 8


def _ln(x, g, b):
    mu = jnp.mean(x, axis=-1, keepdims=True)
    var = jnp.mean((x - mu) ** 2, axis=-1, keepdims=True)
    return (x - mu) * jax.lax.rsqrt(var + 1e-5) * g + b


def _dot(a, b):
    return jnp.dot(a, b, preferred_element_type=jnp.float32)


def _dot16(a, b):
    return jnp.dot(a.astype(jnp.bfloat16), b.astype(jnp.bfloat16),
                   preferred_element_type=jnp.float32)


def _shift_down(x):
    return jnp.concatenate([jnp.zeros_like(x[:1]), x[:-1]], axis=0)


def _shift_up(x):
    return jnp.concatenate([x[1:], jnp.zeros_like(x[:1])], axis=0)


def _attn_block2(hs, ws_list, sel, selT):
    t, d = hs[0].shape
    dh = d // _NUM_HEADS
    scale = 1.0 / math.sqrt(dh)
    qkv = [_dot16(h, w[0]) for h, w in zip(hs, ws_list)]
    q = [x[:, :d] for x in qkv]
    k = [x[:, d:2 * d] for x in qkv]
    v = [x[:, 2 * d:] for x in qkv]
    k_m1 = [_shift_down(x) for x in k]
    k_p1 = [_shift_up(x) for x in k]
    v_m1 = [_shift_down(x) for x in v]
    v_p1 = [_shift_up(x) for x in v]
    ls = [_dot(a * b, sel) * scale for a, b in zip(q, k)]
    ll = [_dot(a * b, sel) * scale for a, b in zip(q, k_m1)]
    lr = [_dot(a * b, sel) * scale for a, b in zip(q, k_p1)]
    row = jax.lax.broadcasted_iota(jnp.int32, ls[0].shape, 0)
    neg = jnp.float32(-1e30)
    ll = [jnp.where(row == 0, neg, x) for x in ll]
    lr = [jnp.where(row == t - 1, neg, x) for x in lr]
    m = [jnp.maximum(a, jnp.maximum(b, c)) for a, b, c in zip(ls, ll, lr)]
    es = [jnp.exp(a - b) for a, b in zip(ls, m)]
    el = [jnp.exp(a - b) for a, b in zip(ll, m)]
    er = [jnp.exp(a - b) for a, b in zip(lr, m)]
    inv = [1.0 / (a + b + c + 1e-9) for a, b, c in zip(es, el, er)]
    wgt_s = [_dot(a * b, selT) for a, b in zip(es, inv)]
    wgt_l = [_dot(a * b, selT) for a, b in zip(el, inv)]
    wgt_r = [_dot(a * b, selT) for a, b in zip(er, inv)]
    out = [a * b + c * e + f * g for a, b, c, e, f, g in
           zip(wgt_s, v, wgt_l, v_m1, wgt_r, v_p1)]
    y = [_dot16(o, w[1]) for o, w in zip(out, ws_list)]
    return [_ln(h + yy, w[2], w[3]) for h, yy, w in zip(hs, y, ws_list)]


def _two_branches(streams, wrefs_list, sel, selT):
    hs = [jax.lax.dot_general(s, w[0][...], (((0,), (0,)), ((), ())),
                              preferred_element_type=jnp.float32) + w[1][...]
          for s, w in zip(streams, wrefs_list)]
    for i in range(3):
        ws_list = [[r[...] for r in w[2 + 4 * i:2 + 4 * i + 4]]
                   for w in wrefs_list]
        hs = _attn_block2(hs, ws_list, sel, selT)
    return [_ln(h, w[14][...], w[15][...]) for h, w in zip(hs, wrefs_list)]


def _body(a_ref, o_ref, *refs):
    wrefs = refs[:-2]
    hg_ref, sensor_ref = refs[-2], refs[-1]

    d = wrefs[0].shape[1]
    dh = d // _NUM_HEADS
    ri = jax.lax.broadcasted_iota(jnp.int32, (d, _NUM_HEADS), 0)
    ci = jax.lax.broadcasted_iota(jnp.int32, (d, _NUM_HEADS), 1)
    sel = (ri // dh == ci).astype(jnp.float32)
    ri2 = jax.lax.broadcasted_iota(jnp.int32, (_NUM_HEADS, d), 0)
    ci2 = jax.lax.broadcasted_iota(jnp.int32, (_NUM_HEADS, d), 1)
    selT = (ci2 // dh == ri2).astype(jnp.float32)

    n_branch = 2 + 3 * 4 + 2
    a_tok, o_tok = _two_branches(
        [a_ref[0], o_ref[0]],
        [wrefs[:n_branch], wrefs[n_branch:2 * n_branch]], sel, selT)

    w1 = wrefs[2 * n_branch][...]
    b1 = wrefs[2 * n_branch + 1][...]
    w2 = wrefs[2 * n_branch + 2][...]
    b2 = wrefs[2 * n_branch + 3][...]
    x = _dot16(a_tok, w1[:d, :]) + _dot16(o_tok, w1[d:, :]) + b1
    x = jax.nn.gelu(x)
    sensor = _dot16(x, w2) + b2
    sensor_ref[0] = sensor
    hg_ref[0] = jnp.mean(sensor, axis=0, keepdims=True)


def _flatten_params(params):
    arrs = []
    for name in ('a', 'omega'):
        p = params[name]
        arrs += [p['in_w'], p['in_b'].reshape(1, -1)]
        for blk in p['blocks']:
            wqkv = jnp.concatenate([blk['wq'], blk['wk'], blk['wv']], axis=1)
            arrs += [wqkv, blk['wo'],
                     blk['ln_g'].reshape(1, -1), blk['ln_b'].reshape(1, -1)]
        arrs += [p['norm_g'].reshape(1, -1), p['norm_b'].reshape(1, -1)]
    arrs += [params['fuse_w1'], params['fuse_b1'].reshape(1, -1),
             params['fuse_w2'], params['fuse_b2'].reshape(1, -1)]
    return arrs


@jax.jit
def kernel(a_stream, omega_stream, params):
    bsz, t, cin = a_stream.shape
    d = params['fuse_w2'].shape[0]
    warrs = _flatten_params(params)
    a_cm = jnp.transpose(a_stream, (0, 2, 1))
    o_cm = jnp.transpose(omega_stream, (0, 2, 1))

    def const_spec(w):
        nd = w.ndim
        return pl.BlockSpec(w.shape, lambda b, _n=nd: (0,) * _n)

    hg, sensor = pl.pallas_call(
        _body,
        grid=(bsz,),
        in_specs=[
            pl.BlockSpec((1, cin, t), lambda b: (b, 0, 0)),
            pl.BlockSpec((1, cin, t), lambda b: (b, 0, 0)),
        ] + [const_spec(w) for w in warrs],
        out_specs=[
            pl.BlockSpec((1, 1, d), lambda b: (b, 0, 0)),
            pl.BlockSpec((1, t, d), lambda b: (b, 0, 0)),
        ],
        out_shape=[
            jax.ShapeDtypeStruct((bsz, 1, d), jnp.float32),
            jax.ShapeDtypeStruct((bsz, t, d), jnp.float32),
        ],
        compiler_params=pltpu.CompilerParams(
            dimension_semantics=("arbitrary",),
            vmem_limit_bytes=120 * 1024 * 1024,
        ),
    )(a_cm, o_cm, *warrs)
    return hg.reshape(bsz, d), sensor


# speedup vs baseline: 1.0860x; 1.0860x over previous
import functools
import math

import jax
import jax.numpy as jnp
from jax.experimental import pallas as pl
from jax.experimental.pallas import tpu as pltpu

_NUM_HEADS = 8


def _ln(x, g, b):
    mu = jnp.mean(x, axis=-1, keepdims=True)
    ms = jnp.mean(x * x, axis=-1, keepdims=True)
    var = ms - mu * mu
    return (x - mu) * jax.lax.rsqrt(var + 1e-5) * g + b


def _dot(a, b):
    return jnp.dot(a, b, preferred_element_type=jnp.float32)


def _dot16(a, b):
    return jnp.dot(a.astype(jnp.bfloat16), b.astype(jnp.bfloat16),
                   preferred_element_type=jnp.float32)


def _shift_down(x):
    return jnp.concatenate([jnp.zeros_like(x[:1]), x[:-1]], axis=0)


def _shift_up(x):
    return jnp.concatenate([x[1:], jnp.zeros_like(x[:1])], axis=0)


def _attn_block2(hs, ws_list, sel, selT):
    t, d = hs[0].shape
    dh = d // _NUM_HEADS
    scale = 1.0 / math.sqrt(dh)
    qkv = [_dot16(h, w[0]).astype(jnp.bfloat16) for h, w in zip(hs, ws_list)]
    q = [x[:, :d] for x in qkv]
    k = [x[:, d:2 * d] for x in qkv]
    v = [x[:, 2 * d:] for x in qkv]
    k_m1 = [_shift_down(x) for x in k]
    k_p1 = [_shift_up(x) for x in k]
    v_m1 = [_shift_down(x) for x in v]
    v_p1 = [_shift_up(x) for x in v]
    sel16 = sel.astype(jnp.bfloat16)
    ls = [_dot(a * b, sel16) * scale for a, b in zip(q, k)]
    ll = [_dot(a * b, sel16) * scale for a, b in zip(q, k_m1)]
    lr = [_dot(a * b, sel16) * scale for a, b in zip(q, k_p1)]
    row = jax.lax.broadcasted_iota(jnp.int32, ls[0].shape, 0)
    neg = jnp.float32(-1e30)
    ll = [jnp.where(row == 0, neg, x) for x in ll]
    lr = [jnp.where(row == t - 1, neg, x) for x in lr]
    m = [jnp.maximum(a, jnp.maximum(b, c)) for a, b, c in zip(ls, ll, lr)]
    es = [jnp.exp(a - b) for a, b in zip(ls, m)]
    el = [jnp.exp(a - b) for a, b in zip(ll, m)]
    er = [jnp.exp(a - b) for a, b in zip(lr, m)]
    inv = [1.0 / (a + b + c + 1e-9) for a, b, c in zip(es, el, er)]
    selT16 = selT.astype(jnp.bfloat16)
    wgt_s = [_dot16(a * b, selT16).astype(jnp.bfloat16) for a, b in zip(es, inv)]
    wgt_l = [_dot16(a * b, selT16).astype(jnp.bfloat16) for a, b in zip(el, inv)]
    wgt_r = [_dot16(a * b, selT16).astype(jnp.bfloat16) for a, b in zip(er, inv)]
    out = [a * b + c * e + f * g for a, b, c, e, f, g in
           zip(wgt_s, v, wgt_l, v_m1, wgt_r, v_p1)]
    y = [_dot16(o, w[1]) for o, w in zip(out, ws_list)]
    return [_ln(h + yy, w[2], w[3]) for h, yy, w in zip(hs, y, ws_list)]


def _two_branches(streams, wrefs_list, sel, selT):
    hs = [jax.lax.dot_general(s, w[0][...], (((0,), (0,)), ((), ())),
                              preferred_element_type=jnp.float32) + w[1][...]
          for s, w in zip(streams, wrefs_list)]
    for i in range(3):
        ws_list = [[r[...] for r in w[2 + 4 * i:2 + 4 * i + 4]]
                   for w in wrefs_list]
        hs = _attn_block2(hs, ws_list, sel, selT)
    return [_ln(h, w[14][...], w[15][...]) for h, w in zip(hs, wrefs_list)]


def _body(a_ref, o_ref, *refs):
    wrefs = refs[:-2]
    hg_ref, sensor_ref = refs[-2], refs[-1]

    d = wrefs[0].shape[1]
    dh = d // _NUM_HEADS
    ri = jax.lax.broadcasted_iota(jnp.int32, (d, _NUM_HEADS), 0)
    ci = jax.lax.broadcasted_iota(jnp.int32, (d, _NUM_HEADS), 1)
    sel = (ri // dh == ci).astype(jnp.float32)
    ri2 = jax.lax.broadcasted_iota(jnp.int32, (_NUM_HEADS, d), 0)
    ci2 = jax.lax.broadcasted_iota(jnp.int32, (_NUM_HEADS, d), 1)
    selT = (ci2 // dh == ri2).astype(jnp.float32)

    n_branch = 2 + 3 * 4 + 2
    a_tok, o_tok = _two_branches(
        [a_ref[0], o_ref[0]],
        [wrefs[:n_branch], wrefs[n_branch:2 * n_branch]], sel, selT)

    w1 = wrefs[2 * n_branch][...]
    b1 = wrefs[2 * n_branch + 1][...]
    w2 = wrefs[2 * n_branch + 2][...]
    b2 = wrefs[2 * n_branch + 3][...]
    x = _dot16(a_tok, w1[:d, :]) + _dot16(o_tok, w1[d:, :]) + b1
    x = jax.nn.gelu(x)
    sensor = _dot16(x, w2) + b2
    sensor_ref[0] = sensor
    hg_ref[0] = jnp.mean(sensor, axis=0, keepdims=True)


def _flatten_params(params):
    arrs = []
    for name in ('a', 'omega'):
        p = params[name]
        arrs += [p['in_w'], p['in_b'].reshape(1, -1)]
        for blk in p['blocks']:
            wqkv = jnp.concatenate([blk['wq'], blk['wk'], blk['wv']], axis=1)
            arrs += [wqkv, blk['wo'],
                     blk['ln_g'].reshape(1, -1), blk['ln_b'].reshape(1, -1)]
        arrs += [p['norm_g'].reshape(1, -1), p['norm_b'].reshape(1, -1)]
    arrs += [params['fuse_w1'], params['fuse_b1'].reshape(1, -1),
             params['fuse_w2'], params['fuse_b2'].reshape(1, -1)]
    return arrs


@jax.jit
def kernel(a_stream, omega_stream, params):
    bsz, t, cin = a_stream.shape
    d = params['fuse_w2'].shape[0]
    warrs = _flatten_params(params)
    a_cm = jnp.transpose(a_stream, (0, 2, 1))
    o_cm = jnp.transpose(omega_stream, (0, 2, 1))

    def const_spec(w):
        nd = w.ndim
        return pl.BlockSpec(w.shape, lambda b, _n=nd: (0,) * _n)

    hg, sensor = pl.pallas_call(
        _body,
        grid=(bsz,),
        in_specs=[
            pl.BlockSpec((1, cin, t), lambda b: (b, 0, 0)),
            pl.BlockSpec((1, cin, t), lambda b: (b, 0, 0)),
        ] + [const_spec(w) for w in warrs],
        out_specs=[
            pl.BlockSpec((1, 1, d), lambda b: (b, 0, 0)),
            pl.BlockSpec((1, t, d), lambda b: (b, 0, 0)),
        ],
        out_shape=[
            jax.ShapeDtypeStruct((bsz, 1, d), jnp.float32),
            jax.ShapeDtypeStruct((bsz, t, d), jnp.float32),
        ],
        compiler_params=pltpu.CompilerParams(
            dimension_semantics=("arbitrary",),
            vmem_limit_bytes=120 * 1024 * 1024,
        ),
    )(a_cm, o_cm, *warrs)
    return hg.reshape(bsz, d), sensor


# speedup vs baseline: 1.0930x; 1.0064x over previous
import functools
import math

import jax
import jax.numpy as jnp
from jax.experimental import pallas as pl
from jax.experimental.pallas import tpu as pltpu

_NUM_HEADS = 8


def _ln(x, g, b):
    mu = jnp.mean(x, axis=-1, keepdims=True)
    ms = jnp.mean(x * x, axis=-1, keepdims=True)
    var = ms - mu * mu
    return (x - mu) * jax.lax.rsqrt(var + 1e-5) * g + b


def _dot(a, b):
    return jnp.dot(a, b, preferred_element_type=jnp.float32)


def _dot16(a, b):
    return jnp.dot(a.astype(jnp.bfloat16), b.astype(jnp.bfloat16),
                   preferred_element_type=jnp.float32)


def _shift_down(x):
    return jnp.concatenate([jnp.zeros_like(x[:1]), x[:-1]], axis=0)


def _shift_up(x):
    return jnp.concatenate([x[1:], jnp.zeros_like(x[:1])], axis=0)


def _attn_block2(hs, ws_list, sel):
    t, d = hs[0].shape
    qkv = [_dot16(h, w[0]).astype(jnp.bfloat16) for h, w in zip(hs, ws_list)]
    q = [x[:, :d] for x in qkv]
    k = [x[:, d:2 * d] for x in qkv]
    v = [x[:, 2 * d:] for x in qkv]
    k_m1 = [_shift_down(x) for x in k]
    k_p1 = [_shift_up(x) for x in k]
    v_m1 = [_shift_down(x) for x in v]
    v_p1 = [_shift_up(x) for x in v]
    nh = _NUM_HEADS
    L = [_dot(a * b, sel[0]) + _dot(a * c, sel[1]) + _dot(a * e, sel[2])
         for a, b, c, e in zip(q, k, k_m1, k_p1)]
    row = jax.lax.broadcasted_iota(jnp.int32, (t, 3 * nh), 0)
    lane = jax.lax.broadcasted_iota(jnp.int32, (t, 3 * nh), 1)
    bad = ((row == 0) & (lane >= nh) & (lane < 2 * nh)) | \
          ((row == t - 1) & (lane >= 2 * nh))
    neg = jnp.float32(-1e30)
    L = [jnp.where(bad, neg, x) for x in L]
    m = [jnp.maximum(x[:, :nh],
                     jnp.maximum(x[:, nh:2 * nh], x[:, 2 * nh:])) for x in L]
    mt = [_dot(x, sel[3]) for x in m]
    e = [jnp.exp(a - b) for a, b in zip(L, mt)]
    den = [_dot(x, sel[4]) for x in e]
    inv = [1.0 / (x + 1e-9) for x in den]
    invt = [_dot(x, sel[3]) for x in inv]
    w24 = [(a * b).astype(jnp.bfloat16) for a, b in zip(e, invt)]
    wexp = [_dot16(x, sel[5]).astype(jnp.bfloat16) for x in w24]
    out = [wx[:, :d] * b + wx[:, d:2 * d] * c + wx[:, 2 * d:] * f
           for wx, b, c, f in zip(wexp, v, v_m1, v_p1)]
    y = [_dot16(o, w[1]) for o, w in zip(out, ws_list)]
    return [_ln(h + yy, w[2], w[3]) for h, yy, w in zip(hs, y, ws_list)]


def _two_branches(streams, wrefs_list, sel):
    hs = [jax.lax.dot_general(s, w[0][...], (((0,), (0,)), ((), ())),
                              preferred_element_type=jnp.float32) + w[1][...]
          for s, w in zip(streams, wrefs_list)]
    for i in range(3):
        ws_list = [[r[...] for r in w[2 + 4 * i:2 + 4 * i + 4]]
                   for w in wrefs_list]
        hs = _attn_block2(hs, ws_list, sel)
    return [_ln(h, w[14][...], w[15][...]) for h, w in zip(hs, wrefs_list)]


def _body(a_ref, o_ref, *refs):
    wrefs = refs[:-2]
    hg_ref, sensor_ref = refs[-2], refs[-1]

    d = wrefs[0].shape[1]
    nh = _NUM_HEADS
    dh = d // nh
    scale = 1.0 / math.sqrt(dh)
    ri = jax.lax.broadcasted_iota(jnp.int32, (d, 3 * nh), 0)
    ci = jax.lax.broadcasted_iota(jnp.int32, (d, 3 * nh), 1)
    same_head = (ci % nh == ri // dh)
    sels = [jnp.where(same_head & (ci // nh == j), scale, 0.0
                      ).astype(jnp.bfloat16) for j in range(3)]
    rb = jax.lax.broadcasted_iota(jnp.int32, (nh, 3 * nh), 0)
    cb = jax.lax.broadcasted_iota(jnp.int32, (nh, 3 * nh), 1)
    bmap = (cb % nh == rb).astype(jnp.float32)
    gmap = bmap.T
    rt = jax.lax.broadcasted_iota(jnp.int32, (3 * nh, 3 * d), 0)
    ct = jax.lax.broadcasted_iota(jnp.int32, (3 * nh, 3 * d), 1)
    t3 = ((ct // d == rt // nh) & ((ct % d) // dh == rt % nh)
          ).astype(jnp.bfloat16)
    sel = [sels[0], sels[1], sels[2], bmap, gmap, t3]

    n_branch = 2 + 3 * 4 + 2
    a_tok, o_tok = _two_branches(
        [a_ref[0], o_ref[0]],
        [wrefs[:n_branch], wrefs[n_branch:2 * n_branch]], sel)

    w1 = wrefs[2 * n_branch][...]
    b1 = wrefs[2 * n_branch + 1][...]
    w2 = wrefs[2 * n_branch + 2][...]
    b2 = wrefs[2 * n_branch + 3][...]
    x = _dot16(a_tok, w1[:d, :]) + _dot16(o_tok, w1[d:, :]) + b1
    x = jax.nn.gelu(x)
    sensor = _dot16(x, w2) + b2
    sensor_ref[0] = sensor
    hg_ref[0] = jnp.mean(sensor, axis=0, keepdims=True)


def _flatten_params(params):
    arrs = []
    for name in ('a', 'omega'):
        p = params[name]
        arrs += [p['in_w'], p['in_b'].reshape(1, -1)]
        for blk in p['blocks']:
            wqkv = jnp.concatenate([blk['wq'], blk['wk'], blk['wv']], axis=1)
            arrs += [wqkv, blk['wo'],
                     blk['ln_g'].reshape(1, -1), blk['ln_b'].reshape(1, -1)]
        arrs += [p['norm_g'].reshape(1, -1), p['norm_b'].reshape(1, -1)]
    arrs += [params['fuse_w1'], params['fuse_b1'].reshape(1, -1),
             params['fuse_w2'], params['fuse_b2'].reshape(1, -1)]
    return arrs


@jax.jit
def kernel(a_stream, omega_stream, params):
    bsz, t, cin = a_stream.shape
    d = params['fuse_w2'].shape[0]
    warrs = _flatten_params(params)
    a_cm = jnp.transpose(a_stream, (0, 2, 1))
    o_cm = jnp.transpose(omega_stream, (0, 2, 1))

    def const_spec(w):
        nd = w.ndim
        return pl.BlockSpec(w.shape, lambda b, _n=nd: (0,) * _n)

    hg, sensor = pl.pallas_call(
        _body,
        grid=(bsz,),
        in_specs=[
            pl.BlockSpec((1, cin, t), lambda b: (b, 0, 0)),
            pl.BlockSpec((1, cin, t), lambda b: (b, 0, 0)),
        ] + [const_spec(w) for w in warrs],
        out_specs=[
            pl.BlockSpec((1, 1, d), lambda b: (b, 0, 0)),
            pl.BlockSpec((1, t, d), lambda b: (b, 0, 0)),
        ],
        out_shape=[
            jax.ShapeDtypeStruct((bsz, 1, d), jnp.float32),
            jax.ShapeDtypeStruct((bsz, t, d), jnp.float32),
        ],
        compiler_params=pltpu.CompilerParams(
            dimension_semantics=("arbitrary",),
            vmem_limit_bytes=120 * 1024 * 1024,
        ),
    )(a_cm, o_cm, *warrs)
    return hg.reshape(bsz, d), sensor
